# baseline (speedup 1.0000x reference)
import functools

import jax
import jax.numpy as jnp
import numpy as np
from jax import lax
from jax.experimental import pallas as pl
from jax.experimental.pallas import tpu as pltpu
from jax.experimental.pallas import tpu_sc as plsc

NUM_LAYER = 5
DIM = 128
N_NODES = 10000
N_EDGES = 320000

NC = 2
NS = 16
NW = NC * NS

CHUNK = 128
DEPTH = 2
CHUNKS_PER_W = 80
E_PAD = NW * CHUNKS_PER_W * CHUNK
WAVES = CHUNKS_PER_W // DEPTH
NPAD = 10240
ROWS_PER_TILE = NPAD // NS
NB = 5
RB = N_NODES // NB

_CT = np.zeros((16, DIM), np.float32)
for _a in range(3):
    for _d in range(3):
        _CT[_a * 4 + _d, _a] = 1.0
        _CT[_a * 4 + _d, 8 + _d] = 1.0


def _zero_tile_chunk(buf, shared, s, ncols):
    z = jnp.zeros((16,), jnp.float32)

    def zrow(i, carry):
        for j in range(ncols // 16):
            buf[i, pl.ds(j * 16, 16)] = z
        return carry

    lax.fori_loop(0, CHUNK, zrow, 0)
    rbase = s * ROWS_PER_TILE
    for k in range(ROWS_PER_TILE // CHUNK):
        pltpu.sync_copy(buf, shared.at[pl.ds(rbase + k * CHUNK, CHUNK)])


def _gather_scatter_body(tab_hbm, idx_hbm, dst_hbm, out_hbm,
                         acc_sh, sbuf, d0, d1, d2, d3,
                         rows, gsem, ssem, xsem):
    c = lax.axis_index("c")
    s = lax.axis_index("s")
    wid = c * NS + s
    dbufs = (d0, d1, d2, d3)

    _zero_tile_chunk(rows.at[0], acc_sh, s, DIM)
    plsc.subcore_barrier()

    nloc = CHUNKS_PER_W * CHUNK

    def wave(g, carry):
        base = wid * nloc + g * (DEPTH * CHUNK)
        xds = [pltpu.async_copy(
            idx_hbm.at[pl.ds(base, DEPTH * CHUNK)], sbuf, xsem)]
        for b in range(DEPTH):
            xds.append(pltpu.async_copy(
                dst_hbm.at[pl.ds(base + b * CHUNK, CHUNK)], dbufs[b], xsem))
        for d in xds:
            d.wait()
        gds = []
        for b in range(DEPTH):
            idx = sbuf.at[pl.ds(b * CHUNK, CHUNK)]
            gds.append(pltpu.async_copy(tab_hbm.at[idx], rows.at[b], gsem))
        for d in gds:
            d.wait()
        sds = []
        for b in range(DEPTH):
            sds.append(pltpu.async_copy(
                rows.at[b], acc_sh.at[dbufs[b]], ssem, add=True))
        for d in sds:
            d.wait()
        return carry

    lax.fori_loop(0, WAVES, wave, 0)
    plsc.subcore_barrier()

    rbase = s * ROWS_PER_TILE
    obase = c * NPAD + rbase
    for k in range(ROWS_PER_TILE // CHUNK):
        pltpu.sync_copy(acc_sh.at[pl.ds(rbase + k * CHUNK, CHUNK)], rows.at[0])
        pltpu.sync_copy(rows.at[0], out_hbm.at[pl.ds(obase + k * CHUNK, CHUNK)])


@functools.lru_cache(maxsize=None)
def _sc_kernels():
    mesh = plsc.VectorSubcoreMesh(
        core_axis_name="c", subcore_axis_name="s",
        num_cores=NC, num_subcores=NS)
    f32 = jnp.float32
    scratch = [
        pltpu.VMEM_SHARED((NPAD, DIM), f32),
        pltpu.VMEM((DEPTH * CHUNK,), jnp.int32),
        pltpu.VMEM((CHUNK,), jnp.int32),
        pltpu.VMEM((CHUNK,), jnp.int32),
        pltpu.VMEM((CHUNK,), jnp.int32),
        pltpu.VMEM((CHUNK,), jnp.int32),
        pltpu.VMEM((DEPTH, CHUNK, DIM), f32),
        pltpu.SemaphoreType.DMA,
        pltpu.SemaphoreType.DMA,
        pltpu.SemaphoreType.DMA,
    ]
    edge_agg = pl.kernel(
        _gather_scatter_body,
        out_type=jax.ShapeDtypeStruct((NC * NPAD, DIM), f32),
        mesh=mesh,
        scratch_types=scratch,
    )
    counts = pl.kernel(
        _gather_scatter_body,
        out_type=jax.ShapeDtypeStruct((NC * NPAD, DIM), f32),
        mesh=mesh,
        scratch_types=scratch,
    )
    return edge_agg, counts


def _lin_body(x_ref, w_ref, b_ref, o_ref):
    o_ref[...] = jnp.maximum(x_ref[...] @ w_ref[...] + b_ref[...], 0.0)


def _mlp_body(p_ref, h_ref, cnt_ref, t_ref, eb_ref,
              w1_ref, b1_ref, w2_ref, b2_ref, o_ref, st_ref):
    i = pl.program_id(0)
    cnt = cnt_ref[0][:, :16] + cnt_ref[1][:, :16]
    emb_agg = jnp.dot(cnt, t_ref[...], precision=lax.Precision.HIGHEST)
    agg = p_ref[0] + p_ref[1] + h_ref[...] + emb_agg + eb_ref[...]
    hid = jnp.maximum(agg @ w1_ref[...] + b1_ref[...], 0.0)
    out = hid @ w2_ref[...] + b2_ref[...]
    o_ref[...] = out
    blk = jnp.stack([jnp.sum(out, axis=0), jnp.sum(out * out, axis=0)])

    @pl.when(i == 0)
    def _():
        st_ref[...] = blk

    @pl.when(i > 0)
    def _():
        st_ref[...] += blk


def _bn_body(o_ref, st_ref, g_ref, be_ref, h_ref):
    inv_n = 1.0 / N_NODES
    mu = st_ref[0:1, :] * inv_n
    var = st_ref[1:2, :] * inv_n - mu * mu
    h_ref[...] = (g_ref[...] * (o_ref[...] - mu) * lax.rsqrt(var + 1e-5)
                  + be_ref[...])


@functools.lru_cache(maxsize=None)
def _tc_kernels():
    f32 = jnp.float32
    lin = pl.pallas_call(
        _lin_body,
        out_shape=jax.ShapeDtypeStruct((N_NODES, DIM), f32),
    )
    mlp = pl.pallas_call(
        _mlp_body,
        grid=(NB,),
        in_specs=[
            pl.BlockSpec((NC, RB, DIM), lambda i: (0, i, 0)),
            pl.BlockSpec((RB, DIM), lambda i: (i, 0)),
            pl.BlockSpec((NC, RB, DIM), lambda i: (0, i, 0)),
            pl.BlockSpec((16, DIM), lambda i: (0, 0)),
            pl.BlockSpec((1, DIM), lambda i: (0, 0)),
            pl.BlockSpec((DIM, 2 * DIM), lambda i: (0, 0)),
            pl.BlockSpec((1, 2 * DIM), lambda i: (0, 0)),
            pl.BlockSpec((2 * DIM, DIM), lambda i: (0, 0)),
            pl.BlockSpec((1, DIM), lambda i: (0, 0)),
        ],
        out_specs=[
            pl.BlockSpec((RB, DIM), lambda i: (i, 0)),
            pl.BlockSpec((2, DIM), lambda i: (0, 0)),
        ],
        out_shape=[
            jax.ShapeDtypeStruct((N_NODES, DIM), f32),
            jax.ShapeDtypeStruct((2, DIM), f32),
        ],
    )
    bn = pl.pallas_call(
        _bn_body,
        grid=(NB,),
        in_specs=[
            pl.BlockSpec((RB, DIM), lambda i: (i, 0)),
            pl.BlockSpec((2, DIM), lambda i: (0, 0)),
            pl.BlockSpec((1, DIM), lambda i: (0, 0)),
            pl.BlockSpec((1, DIM), lambda i: (0, 0)),
        ],
        out_specs=pl.BlockSpec((RB, DIM), lambda i: (i, 0)),
        out_shape=jax.ShapeDtypeStruct((N_NODES, DIM), f32),
    )
    return lin, mlp, bn


def kernel(x, edge_index, edge_attr, lin_W, lin_b, W1, b1, W2, b2,
           emb1, emb2, gamma, beta):
    f32 = jnp.float32
    i32 = jnp.int32
    pad = E_PAD - N_EDGES

    src_p = jnp.concatenate([edge_index[0], jnp.zeros((pad,), i32)])
    pad_dst = N_NODES + jnp.arange(pad, dtype=i32) % (NPAD - N_NODES)
    dst_p = jnp.concatenate([edge_index[1], pad_dst])
    eab_p = jnp.concatenate([edge_attr[:, 0] * 4 + edge_attr[:, 1],
                             jnp.full((pad,), 15, i32)])
    eab_p = eab_p + 16 * (jnp.arange(E_PAD, dtype=i32) % 128)
    tbl = jnp.asarray(np.tile(_CT, (128, 1)))

    edge_agg, counts_k = _sc_kernels()
    lin, mlp, bn = _tc_kernels()

    h = lin(x, lin_W, lin_b.reshape(1, DIM))
    cnt = counts_k(tbl, eab_p, dst_p).reshape(NC, NPAD, DIM)

    for l in range(NUM_LAYER):
        t_l = jnp.concatenate(
            [emb1[l], jnp.zeros((2, DIM), f32), emb2[l],
             jnp.zeros((5, DIM), f32)], axis=0)
        eb_l = (emb1[l, 4] + emb2[l, 0]).reshape(1, DIM)
        p = edge_agg(h, src_p, dst_p).reshape(NC, NPAD, DIM)
        out, st = mlp(p, h, cnt, t_l, eb_l,
                      W1[l], b1[l].reshape(1, 2 * DIM),
                      W2[l], b2[l].reshape(1, DIM))
        h = bn(out, st, gamma[l].reshape(1, DIM), beta[l].reshape(1, DIM))
    return h


# speedup vs baseline: 1.0866x; 1.0866x over previous
import functools

import jax
import jax.numpy as jnp
import numpy as np
from jax import lax
from jax.experimental import pallas as pl
from jax.experimental.pallas import tpu as pltpu
from jax.experimental.pallas import tpu_sc as plsc

NUM_LAYER = 5
DIM = 128
N_NODES = 10000
N_EDGES = 320000

NC = 2
NS = 16
NW = NC * NS

CHUNK = 80
DEPTH = 4
CHUNKS_PER_W = 128
E_PAD = NW * CHUNKS_PER_W * CHUNK
WAVES = CHUNKS_PER_W // DEPTH
NPAD = 10240
ROWS_PER_TILE = NPAD // NS
NB = 5
RB = N_NODES // NB

_CT = np.zeros((16, DIM), np.float32)
for _a in range(3):
    for _d in range(3):
        _CT[_a * 4 + _d, _a] = 1.0
        _CT[_a * 4 + _d, 8 + _d] = 1.0


def _zero_tile_chunk(buf, shared, s, ncols):
    z = jnp.zeros((16,), jnp.float32)

    def zrow(i, carry):
        for j in range(ncols // 16):
            buf[i, pl.ds(j * 16, 16)] = z
        return carry

    lax.fori_loop(0, CHUNK, zrow, 0)
    rbase = s * ROWS_PER_TILE
    for k in range(ROWS_PER_TILE // CHUNK):
        pltpu.sync_copy(buf, shared.at[pl.ds(rbase + k * CHUNK, CHUNK)])


def _gather_scatter_body(tab_hbm, idx_hbm, dst_hbm, out_hbm,
                         acc_sh, sbuf, d0, d1, d2, d3,
                         rows, gsem, ssem, xsem):
    c = lax.axis_index("c")
    s = lax.axis_index("s")
    wid = c * NS + s
    dbufs = (d0, d1, d2, d3)

    _zero_tile_chunk(rows.at[0], acc_sh, s, DIM)
    plsc.subcore_barrier()

    nloc = CHUNKS_PER_W * CHUNK

    def wave(g, carry):
        base = wid * nloc + g * (DEPTH * CHUNK)
        xds = [pltpu.async_copy(
            idx_hbm.at[pl.ds(base, DEPTH * CHUNK)], sbuf, xsem)]
        for b in range(DEPTH):
            xds.append(pltpu.async_copy(
                dst_hbm.at[pl.ds(base + b * CHUNK, CHUNK)], dbufs[b], xsem))
        for d in xds:
            d.wait()
        gds = []
        for b in range(DEPTH):
            idx = sbuf.at[pl.ds(b * CHUNK, CHUNK)]
            gds.append(pltpu.async_copy(tab_hbm.at[idx], rows.at[b], gsem))
        sds = []
        for b in range(DEPTH):
            gds[b].wait()
            sds.append(pltpu.async_copy(
                rows.at[b], acc_sh.at[dbufs[b]], ssem, add=True))
        for d in sds:
            d.wait()
        return carry

    lax.fori_loop(0, WAVES, wave, 0)
    plsc.subcore_barrier()

    rbase = s * ROWS_PER_TILE
    obase = c * NPAD + rbase
    for k in range(ROWS_PER_TILE // CHUNK):
        pltpu.sync_copy(acc_sh.at[pl.ds(rbase + k * CHUNK, CHUNK)], rows.at[0])
        pltpu.sync_copy(rows.at[0], out_hbm.at[pl.ds(obase + k * CHUNK, CHUNK)])


@functools.lru_cache(maxsize=None)
def _sc_kernels():
    mesh = plsc.VectorSubcoreMesh(
        core_axis_name="c", subcore_axis_name="s",
        num_cores=NC, num_subcores=NS)
    f32 = jnp.float32
    scratch = [
        pltpu.VMEM_SHARED((NPAD, DIM), f32),
        pltpu.VMEM((DEPTH * CHUNK,), jnp.int32),
        pltpu.VMEM((CHUNK,), jnp.int32),
        pltpu.VMEM((CHUNK,), jnp.int32),
        pltpu.VMEM((CHUNK,), jnp.int32),
        pltpu.VMEM((CHUNK,), jnp.int32),
        pltpu.VMEM((DEPTH, CHUNK, DIM), f32),
        pltpu.SemaphoreType.DMA,
        pltpu.SemaphoreType.DMA,
        pltpu.SemaphoreType.DMA,
    ]
    edge_agg = pl.kernel(
        _gather_scatter_body,
        out_type=jax.ShapeDtypeStruct((NC * NPAD, DIM), f32),
        mesh=mesh,
        scratch_types=scratch,
    )
    counts = pl.kernel(
        _gather_scatter_body,
        out_type=jax.ShapeDtypeStruct((NC * NPAD, DIM), f32),
        mesh=mesh,
        scratch_types=scratch,
    )
    return edge_agg, counts


def _lin_body(x_ref, w_ref, b_ref, o_ref):
    o_ref[...] = jnp.maximum(x_ref[...] @ w_ref[...] + b_ref[...], 0.0)


def _mlp_body(p_ref, h_ref, cnt_ref, t_ref, eb_ref,
              w1_ref, b1_ref, w2_ref, b2_ref, o_ref, st_ref):
    i = pl.program_id(0)
    cnt = cnt_ref[0][:, :16] + cnt_ref[1][:, :16]
    emb_agg = jnp.dot(cnt, t_ref[...], precision=lax.Precision.HIGHEST)
    agg = p_ref[0] + p_ref[1] + h_ref[...] + emb_agg + eb_ref[...]
    hid = jnp.maximum(agg @ w1_ref[...] + b1_ref[...], 0.0)
    out = hid @ w2_ref[...] + b2_ref[...]
    o_ref[...] = out
    blk = jnp.stack([jnp.sum(out, axis=0), jnp.sum(out * out, axis=0)])

    @pl.when(i == 0)
    def _():
        st_ref[...] = blk

    @pl.when(i > 0)
    def _():
        st_ref[...] += blk


def _bn_body(o_ref, st_ref, g_ref, be_ref, h_ref):
    inv_n = 1.0 / N_NODES
    mu = st_ref[0:1, :] * inv_n
    var = st_ref[1:2, :] * inv_n - mu * mu
    h_ref[...] = (g_ref[...] * (o_ref[...] - mu) * lax.rsqrt(var + 1e-5)
                  + be_ref[...])


@functools.lru_cache(maxsize=None)
def _tc_kernels():
    f32 = jnp.float32
    lin = pl.pallas_call(
        _lin_body,
        out_shape=jax.ShapeDtypeStruct((N_NODES, DIM), f32),
    )
    mlp = pl.pallas_call(
        _mlp_body,
        grid=(NB,),
        in_specs=[
            pl.BlockSpec((NC, RB, DIM), lambda i: (0, i, 0)),
            pl.BlockSpec((RB, DIM), lambda i: (i, 0)),
            pl.BlockSpec((NC, RB, DIM), lambda i: (0, i, 0)),
            pl.BlockSpec((16, DIM), lambda i: (0, 0)),
            pl.BlockSpec((1, DIM), lambda i: (0, 0)),
            pl.BlockSpec((DIM, 2 * DIM), lambda i: (0, 0)),
            pl.BlockSpec((1, 2 * DIM), lambda i: (0, 0)),
            pl.BlockSpec((2 * DIM, DIM), lambda i: (0, 0)),
            pl.BlockSpec((1, DIM), lambda i: (0, 0)),
        ],
        out_specs=[
            pl.BlockSpec((RB, DIM), lambda i: (i, 0)),
            pl.BlockSpec((2, DIM), lambda i: (0, 0)),
        ],
        out_shape=[
            jax.ShapeDtypeStruct((N_NODES, DIM), f32),
            jax.ShapeDtypeStruct((2, DIM), f32),
        ],
    )
    bn = pl.pallas_call(
        _bn_body,
        grid=(NB,),
        in_specs=[
            pl.BlockSpec((RB, DIM), lambda i: (i, 0)),
            pl.BlockSpec((2, DIM), lambda i: (0, 0)),
            pl.BlockSpec((1, DIM), lambda i: (0, 0)),
            pl.BlockSpec((1, DIM), lambda i: (0, 0)),
        ],
        out_specs=pl.BlockSpec((RB, DIM), lambda i: (i, 0)),
        out_shape=jax.ShapeDtypeStruct((N_NODES, DIM), f32),
    )
    return lin, mlp, bn


def kernel(x, edge_index, edge_attr, lin_W, lin_b, W1, b1, W2, b2,
           emb1, emb2, gamma, beta):
    f32 = jnp.float32
    i32 = jnp.int32
    pad = E_PAD - N_EDGES

    src_p = jnp.concatenate([edge_index[0], jnp.zeros((pad,), i32)])
    pad_dst = N_NODES + jnp.arange(pad, dtype=i32) % (NPAD - N_NODES)
    dst_p = jnp.concatenate([edge_index[1], pad_dst])
    eab_p = jnp.concatenate([edge_attr[:, 0] * 4 + edge_attr[:, 1],
                             jnp.full((pad,), 15, i32)])
    eab_p = eab_p + 16 * (jnp.arange(E_PAD, dtype=i32) % 128)
    tbl = jnp.asarray(np.tile(_CT, (128, 1)))

    edge_agg, counts_k = _sc_kernels()
    lin, mlp, bn = _tc_kernels()

    h = lin(x, lin_W, lin_b.reshape(1, DIM))
    cnt = counts_k(tbl, eab_p, dst_p).reshape(NC, NPAD, DIM)

    for l in range(NUM_LAYER):
        t_l = jnp.concatenate(
            [emb1[l], jnp.zeros((2, DIM), f32), emb2[l],
             jnp.zeros((5, DIM), f32)], axis=0)
        eb_l = (emb1[l, 4] + emb2[l, 0]).reshape(1, DIM)
        p = edge_agg(h, src_p, dst_p).reshape(NC, NPAD, DIM)
        out, st = mlp(p, h, cnt, t_l, eb_l,
                      W1[l], b1[l].reshape(1, 2 * DIM),
                      W2[l], b2[l].reshape(1, DIM))
        h = bn(out, st, gamma[l].reshape(1, DIM), beta[l].reshape(1, DIM))
    return h


# speedup vs baseline: 1.1148x; 1.0259x over previous
import functools

import jax
import jax.numpy as jnp
import numpy as np
from jax import lax
from jax.experimental import pallas as pl
from jax.experimental.pallas import tpu as pltpu
from jax.experimental.pallas import tpu_sc as plsc

NUM_LAYER = 5
DIM = 128
N_NODES = 10000
N_EDGES = 320000

NC = 2
NS = 16
NW = NC * NS

CHUNK = 80
DEPTH = 4
CHUNKS_PER_W = 128
E_PAD = NW * CHUNKS_PER_W * CHUNK
WAVES = CHUNKS_PER_W // DEPTH
NPAD = 10240
ROWS_PER_TILE = NPAD // NS
NB = 5
RB = N_NODES // NB

_CT = np.zeros((16, DIM), np.float32)
for _a in range(3):
    for _d in range(3):
        _CT[_a * 4 + _d, _a] = 1.0
        _CT[_a * 4 + _d, 8 + _d] = 1.0


def _zero_tile_chunk(buf, shared, s, ncols):
    z = jnp.zeros((16,), jnp.float32)

    def zrow(i, carry):
        for j in range(ncols // 16):
            buf[i, pl.ds(j * 16, 16)] = z
        return carry

    lax.fori_loop(0, CHUNK, zrow, 0)
    rbase = s * ROWS_PER_TILE
    for k in range(ROWS_PER_TILE // CHUNK):
        pltpu.sync_copy(buf, shared.at[pl.ds(rbase + k * CHUNK, CHUNK)])


def _gather_scatter_body(tab_hbm, idx_hbm, dst_hbm, out_hbm,
                         acc_sh, sb0, sb1, d0, d1, d2, d3, d4, d5, d6, d7,
                         rows, gsem, ssem, xsem):
    c = lax.axis_index("c")
    s = lax.axis_index("s")
    wid = c * NS + s
    sets = ((sb0, (d0, d1, d2, d3)), (sb1, (d4, d5, d6, d7)))

    _zero_tile_chunk(rows.at[0], acc_sh, s, DIM)
    plsc.subcore_barrier()

    nloc = CHUNKS_PER_W * CHUNK
    wbase = wid * nloc

    def stage(w, sb, dbs):
        base = wbase + w * (DEPTH * CHUNK)
        ds = [pltpu.async_copy(
            idx_hbm.at[pl.ds(base, DEPTH * CHUNK)], sb, xsem)]
        for b in range(DEPTH):
            ds.append(pltpu.async_copy(
                dst_hbm.at[pl.ds(base + b * CHUNK, CHUNK)], dbs[b], xsem))
        return ds

    def run_wave(sb, dbs):
        gds = []
        for b in range(DEPTH):
            idx = sb.at[pl.ds(b * CHUNK, CHUNK)]
            gds.append(pltpu.async_copy(tab_hbm.at[idx], rows.at[b], gsem))
        sds = []
        for b in range(DEPTH):
            gds[b].wait()
            sds.append(pltpu.async_copy(
                rows.at[b], acc_sh.at[dbs[b]], ssem, add=True))
        for d in sds:
            d.wait()

    for d in stage(0, *sets[0]):
        d.wait()

    def pair(gp, carry):
        ds1 = stage(2 * gp + 1, *sets[1])
        run_wave(*sets[0])
        for d in ds1:
            d.wait()
        ds0 = stage(jnp.minimum(2 * gp + 2, WAVES - 1), *sets[0])
        run_wave(*sets[1])
        for d in ds0:
            d.wait()
        return carry

    lax.fori_loop(0, WAVES // 2, pair, 0)
    plsc.subcore_barrier()

    rbase = s * ROWS_PER_TILE
    obase = c * NPAD + rbase
    for k in range(ROWS_PER_TILE // CHUNK):
        pltpu.sync_copy(acc_sh.at[pl.ds(rbase + k * CHUNK, CHUNK)], rows.at[0])
        pltpu.sync_copy(rows.at[0], out_hbm.at[pl.ds(obase + k * CHUNK, CHUNK)])


@functools.lru_cache(maxsize=None)
def _sc_kernels():
    mesh = plsc.VectorSubcoreMesh(
        core_axis_name="c", subcore_axis_name="s",
        num_cores=NC, num_subcores=NS)
    f32 = jnp.float32
    scratch = (
        [pltpu.VMEM_SHARED((NPAD, DIM), f32)]
        + [pltpu.VMEM((DEPTH * CHUNK,), jnp.int32)] * 2
        + [pltpu.VMEM((CHUNK,), jnp.int32)] * 8
        + [pltpu.VMEM((DEPTH, CHUNK, DIM), f32)]
        + [pltpu.SemaphoreType.DMA] * 3
    )
    edge_agg = pl.kernel(
        _gather_scatter_body,
        out_type=jax.ShapeDtypeStruct((NC * NPAD, DIM), f32),
        mesh=mesh,
        scratch_types=scratch,
    )
    counts = pl.kernel(
        _gather_scatter_body,
        out_type=jax.ShapeDtypeStruct((NC * NPAD, DIM), f32),
        mesh=mesh,
        scratch_types=scratch,
    )
    return edge_agg, counts


def _lin_body(x_ref, w_ref, b_ref, o_ref):
    o_ref[...] = jnp.maximum(x_ref[...] @ w_ref[...] + b_ref[...], 0.0)


def _mlp_body(p_ref, h_ref, cnt_ref, t_ref, eb_ref,
              w1_ref, b1_ref, w2_ref, b2_ref, o_ref, st_ref):
    i = pl.program_id(0)
    cnt = cnt_ref[0][:, :16] + cnt_ref[1][:, :16]
    emb_agg = jnp.dot(cnt, t_ref[...], precision=lax.Precision.HIGHEST)
    agg = p_ref[0] + p_ref[1] + h_ref[...] + emb_agg + eb_ref[...]
    hid = jnp.maximum(agg @ w1_ref[...] + b1_ref[...], 0.0)
    out = hid @ w2_ref[...] + b2_ref[...]
    o_ref[...] = out
    blk = jnp.stack([jnp.sum(out, axis=0), jnp.sum(out * out, axis=0)])

    @pl.when(i == 0)
    def _():
        st_ref[...] = blk

    @pl.when(i > 0)
    def _():
        st_ref[...] += blk


def _bn_body(o_ref, st_ref, g_ref, be_ref, h_ref):
    inv_n = 1.0 / N_NODES
    mu = st_ref[0:1, :] * inv_n
    var = st_ref[1:2, :] * inv_n - mu * mu
    h_ref[...] = (g_ref[...] * (o_ref[...] - mu) * lax.rsqrt(var + 1e-5)
                  + be_ref[...])


@functools.lru_cache(maxsize=None)
def _tc_kernels():
    f32 = jnp.float32
    lin = pl.pallas_call(
        _lin_body,
        out_shape=jax.ShapeDtypeStruct((N_NODES, DIM), f32),
    )
    mlp = pl.pallas_call(
        _mlp_body,
        grid=(NB,),
        in_specs=[
            pl.BlockSpec((NC, RB, DIM), lambda i: (0, i, 0)),
            pl.BlockSpec((RB, DIM), lambda i: (i, 0)),
            pl.BlockSpec((NC, RB, DIM), lambda i: (0, i, 0)),
            pl.BlockSpec((16, DIM), lambda i: (0, 0)),
            pl.BlockSpec((1, DIM), lambda i: (0, 0)),
            pl.BlockSpec((DIM, 2 * DIM), lambda i: (0, 0)),
            pl.BlockSpec((1, 2 * DIM), lambda i: (0, 0)),
            pl.BlockSpec((2 * DIM, DIM), lambda i: (0, 0)),
            pl.BlockSpec((1, DIM), lambda i: (0, 0)),
        ],
        out_specs=[
            pl.BlockSpec((RB, DIM), lambda i: (i, 0)),
            pl.BlockSpec((2, DIM), lambda i: (0, 0)),
        ],
        out_shape=[
            jax.ShapeDtypeStruct((N_NODES, DIM), f32),
            jax.ShapeDtypeStruct((2, DIM), f32),
        ],
    )
    bn = pl.pallas_call(
        _bn_body,
        grid=(NB,),
        in_specs=[
            pl.BlockSpec((RB, DIM), lambda i: (i, 0)),
            pl.BlockSpec((2, DIM), lambda i: (0, 0)),
            pl.BlockSpec((1, DIM), lambda i: (0, 0)),
            pl.BlockSpec((1, DIM), lambda i: (0, 0)),
        ],
        out_specs=pl.BlockSpec((RB, DIM), lambda i: (i, 0)),
        out_shape=jax.ShapeDtypeStruct((N_NODES, DIM), f32),
    )
    return lin, mlp, bn


def kernel(x, edge_index, edge_attr, lin_W, lin_b, W1, b1, W2, b2,
           emb1, emb2, gamma, beta):
    f32 = jnp.float32
    i32 = jnp.int32
    pad = E_PAD - N_EDGES

    src_p = jnp.concatenate([edge_index[0], jnp.zeros((pad,), i32)])
    pad_dst = N_NODES + jnp.arange(pad, dtype=i32) % (NPAD - N_NODES)
    dst_p = jnp.concatenate([edge_index[1], pad_dst])
    eab_p = jnp.concatenate([edge_attr[:, 0] * 4 + edge_attr[:, 1],
                             jnp.full((pad,), 15, i32)])
    eab_p = eab_p + 16 * (jnp.arange(E_PAD, dtype=i32) % 128)
    tbl = jnp.asarray(np.tile(_CT, (128, 1)))

    edge_agg, counts_k = _sc_kernels()
    lin, mlp, bn = _tc_kernels()

    h = lin(x, lin_W, lin_b.reshape(1, DIM))
    cnt = counts_k(tbl, eab_p, dst_p).reshape(NC, NPAD, DIM)

    for l in range(NUM_LAYER):
        t_l = jnp.concatenate(
            [emb1[l], jnp.zeros((2, DIM), f32), emb2[l],
             jnp.zeros((5, DIM), f32)], axis=0)
        eb_l = (emb1[l, 4] + emb2[l, 0]).reshape(1, DIM)
        p = edge_agg(h, src_p, dst_p).reshape(NC, NPAD, DIM)
        out, st = mlp(p, h, cnt, t_l, eb_l,
                      W1[l], b1[l].reshape(1, 2 * DIM),
                      W2[l], b2[l].reshape(1, DIM))
        h = bn(out, st, gamma[l].reshape(1, DIM), beta[l].reshape(1, DIM))
    return h
